# speedup vs baseline: 2.6175x; 2.6175x over previous
import jax
import jax.numpy as jnp
from jax.experimental import pallas as pl

_BLK = 512
_V = 11


def _body(s_ref, x_ref, t_ref, o_ref):
    s = s_ref[0, 0]
    oh = (s[:, None] == jax.lax.broadcasted_iota(jnp.int32, (1, _V), 1)
          ).astype(jnp.float32)
    emb = jnp.dot(oh, t_ref[...], preferred_element_type=jnp.float32)
    o_ref[...] = x_ref[...] + emb


def kernel(x, scores, score_embeddings):
    B, L, D = x.shape
    N = B * L
    xf = x.reshape(N, D)
    sf = scores.reshape(N // _BLK, 1, _BLK).astype(jnp.int32)
    out = pl.pallas_call(
        _body,
        grid=(N // _BLK,),
        in_specs=[
            pl.BlockSpec((1, 1, _BLK), lambda i: (i, 0, 0)),
            pl.BlockSpec((_BLK, D), lambda i: (i, 0)),
            pl.BlockSpec((_V, D), lambda i: (0, 0)),
        ],
        out_specs=pl.BlockSpec((_BLK, D), lambda i: (i, 0)),
        out_shape=jax.ShapeDtypeStruct((N, D), jnp.float32),
    )(sf, xf, score_embeddings)
    return out.reshape(B, L, D)
